# Initial kernel scaffold; baseline (speedup 1.0000x reference)
import jax
import jax.numpy as jnp
from jax.experimental import pallas as pl


def kernel(x, edge_index, edge_attr, params):
    raise NotImplementedError("write your pallas kernel here")


# speedup vs baseline: 2.7987x; 2.7987x over previous
import functools
import math

import jax
import jax.numpy as jnp
from jax import lax
from jax.experimental import pallas as pl
from jax.experimental.pallas import tpu as pltpu
from jax.experimental.pallas import tpu_sc as plsc

F32 = jnp.float32
I32 = jnp.int32

NC = 2
NS = 16
NW = NC * NS
LANES = 16

D = 128
SHIFT_MARGIN = 60.0


def _proj_body(h_ref, w_ref, b_ref, q_ref, k_ref, v_ref, r_ref):
    pr = jnp.dot(h_ref[...], w_ref[...], preferred_element_type=F32) + b_ref[...]
    q_ref[...] = pr[:, :D]
    k_ref[...] = pr[:, D:2 * D]
    v_ref[...] = pr[:, 2 * D:3 * D]
    r_ref[...] = pr[:, 3 * D:]


def _projections(h, wcat, bcat, n_blk):
    n = h.shape[0]
    grid = n // n_blk
    out = jax.ShapeDtypeStruct((n, D), F32)
    return pl.pallas_call(
        _proj_body,
        grid=(grid,),
        in_specs=[
            pl.BlockSpec((n_blk, D), lambda i: (i, 0)),
            pl.BlockSpec((D, 4 * D), lambda i: (0, 0)),
            pl.BlockSpec((1, 4 * D), lambda i: (0, 0)),
        ],
        out_specs=[pl.BlockSpec((n_blk, D), lambda i: (i, 0))] * 4,
        out_shape=[out, out, out, out],
    )(h, wcat, bcat)


def _scalars_body(q_ref, k_ref, we_ref, be_ref, qwe_ref, qbe2_ref):
    q = q_ref[...]
    k = k_ref[...]
    we = we_ref[...]
    be = be_ref[...]
    qwe = jnp.sum(q * we, axis=1)
    qbe = jnp.sum(q * be, axis=1)
    qn = jnp.sqrt(jnp.sum(q * q, axis=1))
    kn2max = jnp.max(jnp.sum(k * k, axis=1))
    shift = qn * jnp.sqrt(kn2max) - SHIFT_MARGIN
    pad = jnp.zeros((qwe_ref.shape[0] - q.shape[0],), F32)
    qwe_ref[...] = jnp.concatenate([qwe, pad])
    qbe2_ref[...] = jnp.concatenate([qbe - shift, pad])


def _node_scalars(q, k, we, be, n_pad):
    out = jax.ShapeDtypeStruct((n_pad,), F32)
    return pl.pallas_call(_scalars_body, out_shape=[out, out])(q, k, we, be)


def _combine_body(den_ref, u_ref, dinv_ref, satt_ref, t_ref):
    den = jnp.sum(den_ref[...], axis=0)
    u = jnp.sum(u_ref[...], axis=0)
    pos = den > 0.0
    dinv = jnp.where(pos, 1.0 / jnp.where(pos, den, 1.0), 0.0)
    dinv_ref[...] = dinv
    satt_ref[...] = u * dinv
    t_ref[...] = jnp.where(pos, 1.0, 0.0)


def _combine(den_p, u_p, n_pad):
    out = jax.ShapeDtypeStruct((n_pad,), F32)
    return pl.pallas_call(_combine_body, out_shape=[out, out, out])(den_p, u_p)


def _gate_body(op_ref, r_ref, dinv_ref, satt_ref, t_ref, we_ref, be_ref,
               g1_ref, g2_ref, gam_ref, bln_ref, h_ref, *, do_ln):
    n = r_ref.shape[0]
    o = op_ref[0, :n, :] + op_ref[1, :n, :]
    di = dinv_ref[...][:n]
    sa = satt_ref[...][:n]
    tt = t_ref[...][:n]
    o = di[:, None] * o + sa[:, None] * we_ref[...] + tt[:, None] * be_ref[...]
    r = r_ref[...]
    logit = (jnp.sum(o * g1_ref[...], axis=1, keepdims=True)
             + jnp.sum(r * g2_ref[...], axis=1, keepdims=True))
    beta = jax.nn.sigmoid(logit)
    h = beta * r + (1.0 - beta) * o
    if do_ln:
        mu = jnp.mean(h, axis=1, keepdims=True)
        var = jnp.mean((h - mu) ** 2, axis=1, keepdims=True)
        h = (h - mu) / jnp.sqrt(var + 1e-5) * gam_ref[...] + bln_ref[...]
        h = jnp.maximum(h, 0.0)
    h_ref[...] = h


def _gate(out_p, r, dinv, satt, t, we, be, g1, g2, gam, bln, do_ln):
    n = r.shape[0]
    return pl.pallas_call(
        functools.partial(_gate_body, do_ln=do_ln),
        out_shape=jax.ShapeDtypeStruct((n, D), F32),
    )(out_p, r, dinv, satt, t, we, be, g1, g2, gam, bln)


def _sc_mesh():
    return plsc.VectorSubcoreMesh(core_axis_name="c", subcore_axis_name="s")


def _make_pass1(n, n_pad, e, epw, b):
    nchunk = epw // b
    ngrp = b // LANES

    def body(q_hbm, k_hbm, dst_hbm, src_hbm, ea_hbm, qwe_hbm, qbe2_hbm,
             ex_hbm, den_hbm, u_hbm,
             dst_l, src_l, ea_l, ex_l, qwe_l, qbe2_l, den_l, u_l,
             qb0, kb0, qb1, kb1, sq0, sk0, sq1, sk1):
        wid = lax.axis_index("s") * NC + lax.axis_index("c")
        base = wid * epw
        pltpu.sync_copy(dst_hbm.at[pl.ds(base, epw)], dst_l)
        pltpu.sync_copy(src_hbm.at[pl.ds(base, epw)], src_l)
        pltpu.sync_copy(ea_hbm.at[pl.ds(base, epw)], ea_l)
        pltpu.sync_copy(qwe_hbm, qwe_l)
        pltpu.sync_copy(qbe2_hbm, qbe2_l)

        def zbody(i, _):
            z = jnp.zeros((LANES,), F32)
            den_l[pl.ds(i * LANES, LANES)] = z
            u_l[pl.ds(i * LANES, LANES)] = z
            return 0
        lax.fori_loop(0, n_pad // LANES, zbody, 0, unroll=4)

        def start(c, qb, kb, sq, sk):
            pltpu.make_async_copy(
                q_hbm.at[dst_l.at[pl.ds(c * b, b)]], qb, sq).start()
            pltpu.make_async_copy(
                k_hbm.at[src_l.at[pl.ds(c * b, b)]], kb, sk).start()

        def wait(qb, kb, sq, sk):
            pltpu.make_async_copy(q_hbm.at[pl.ds(0, b)], qb, sq).wait()
            pltpu.make_async_copy(k_hbm.at[pl.ds(0, b)], kb, sk).wait()

        def compute(c, qb, kb):
            for g in range(ngrp):
                goff = c * b + g * LANES
                dstg = dst_l[pl.ds(goff, LANES)]
                eag = ea_l[pl.ds(goff, LANES)]
                qweg = plsc.load_gather(qwe_l, [dstg])
                qbe2g = plsc.load_gather(qbe2_l, [dstg])
                ridx = lax.iota(I32, LANES) + g * LANES

                def jbody(j, acc):
                    a = acc
                    for jj in range(4):
                        jc = jnp.full((LANES,), j * 4 + jj, I32)
                        colq = plsc.load_gather(qb, [ridx, jc])
                        colk = plsc.load_gather(kb, [ridx, jc])
                        a = a + colq * colk
                    return a
                acc = lax.fori_loop(0, D // 4, jbody,
                                    jnp.zeros((LANES,), F32))
                ex = jnp.exp(acc + eag * qweg + qbe2g)
                ex_l[pl.ds(goff, LANES)] = ex
                exu = ex * eag
                lane = lax.iota(I32, LANES)
                for i in range(LANES):
                    m = lane == i
                    plsc.addupdate_scatter(den_l, [dstg], ex, mask=m)
                    plsc.addupdate_scatter(u_l, [dstg], exu, mask=m)

        start(0, qb0, kb0, sq0, sk0)
        start(1, qb1, kb1, sq1, sk1)

        def chunk_pair(i, _):
            c = i * 2
            wait(qb0, kb0, sq0, sk0)
            compute(c, qb0, kb0)

            @pl.when(c + 2 < nchunk)
            def _():
                start(c + 2, qb0, kb0, sq0, sk0)
            wait(qb1, kb1, sq1, sk1)
            compute(c + 1, qb1, kb1)

            @pl.when(c + 3 < nchunk)
            def _():
                start(c + 3, qb1, kb1, sq1, sk1)
            return 0
        lax.fori_loop(0, nchunk // 2, chunk_pair, 0)
        if nchunk % 2:
            wait(qb0, kb0, sq0, sk0)
            compute(nchunk - 1, qb0, kb0)

        pltpu.sync_copy(ex_l, ex_hbm.at[pl.ds(base, epw)])
        pltpu.sync_copy(den_l, den_hbm.at[wid])
        pltpu.sync_copy(u_l, u_hbm.at[wid])

    return pl.kernel(
        body,
        out_type=(jax.ShapeDtypeStruct((e,), F32),
                  jax.ShapeDtypeStruct((NW, n_pad), F32),
                  jax.ShapeDtypeStruct((NW, n_pad), F32)),
        mesh=_sc_mesh(),
        compiler_params=pltpu.CompilerParams(needs_layout_passes=False),
        scratch_types=[
            pltpu.VMEM((epw,), I32),
            pltpu.VMEM((epw,), I32),
            pltpu.VMEM((epw,), F32),
            pltpu.VMEM((epw,), F32),
            pltpu.VMEM((n_pad,), F32),
            pltpu.VMEM((n_pad,), F32),
            pltpu.VMEM((n_pad,), F32),
            pltpu.VMEM((n_pad,), F32),
            pltpu.VMEM((b, D), F32),
            pltpu.VMEM((b, D), F32),
            pltpu.VMEM((b, D), F32),
            pltpu.VMEM((b, D), F32),
            pltpu.SemaphoreType.DMA,
            pltpu.SemaphoreType.DMA,
            pltpu.SemaphoreType.DMA,
            pltpu.SemaphoreType.DMA,
        ],
    )


def _make_pass2(n, n_pad, e, epw, b):
    nchunk = epw // b
    ngrp = b // LANES
    rows_per_tile = n_pad // NS

    def body(v_hbm, ex_hbm, dst_hbm, src_hbm,
             outp_hbm,
             dst_l, src_l,
             vb0, vb1, exc0, exc1, sv0, sv1, se0, se1, out_sh):
        cid = lax.axis_index("c")
        sid = lax.axis_index("s")
        wid = sid * NC + cid
        base = wid * epw
        pltpu.sync_copy(dst_hbm.at[pl.ds(base, epw)], dst_l)
        pltpu.sync_copy(src_hbm.at[pl.ds(base, epw)], src_l)

        def zb(i, _):
            vb0[i // (D // LANES), pl.ds((i % (D // LANES)) * LANES, LANES)] = (
                jnp.zeros((LANES,), F32))
            return 0
        lax.fori_loop(0, b * D // LANES, zb, 0, unroll=4)
        for j in range(rows_per_tile // b):
            pltpu.sync_copy(vb0, out_sh.at[pl.ds(sid * rows_per_tile + j * b, b)])
        rem = rows_per_tile % b
        if rem:
            pltpu.sync_copy(
                vb0.at[pl.ds(0, rem)],
                out_sh.at[pl.ds(sid * rows_per_tile + (rows_per_tile // b) * b,
                                rem)])
        plsc.subcore_barrier()

        def start(c, vb, sv, exc, se):
            pltpu.make_async_copy(
                v_hbm.at[src_l.at[pl.ds(c * b, b)]], vb, sv).start()
            pltpu.make_async_copy(
                ex_hbm.at[pl.ds(base + c * b, b)], exc, se).start()

        def wait(vb, sv, exc, se):
            pltpu.make_async_copy(v_hbm.at[pl.ds(0, b)], vb, sv).wait()
            pltpu.make_async_copy(ex_hbm.at[pl.ds(0, b)], exc, se).wait()

        def compute(c, vb, exc):
            ws = [exc[pl.ds(g * LANES, LANES)] for g in range(ngrp)]

            def jbody(j, _):
                for g in range(ngrp):
                    ridx = lax.iota(I32, LANES) + g * LANES
                    jc = jnp.full((LANES,), j, I32)
                    col = plsc.load_gather(vb, [ridx, jc])
                    plsc.store_scatter(vb, [ridx, jc], col * ws[g])
                return 0
            lax.fori_loop(0, D, jbody, 0, unroll=2)
            pltpu.sync_copy(vb, out_sh.at[dst_l.at[pl.ds(c * b, b)]], add=True)

        start(0, vb0, sv0, exc0, se0)
        start(1, vb1, sv1, exc1, se1)

        def chunk_pair(i, _):
            c = i * 2
            wait(vb0, sv0, exc0, se0)
            compute(c, vb0, exc0)

            @pl.when(c + 2 < nchunk)
            def _():
                start(c + 2, vb0, sv0, exc0, se0)
            wait(vb1, sv1, exc1, se1)
            compute(c + 1, vb1, exc1)

            @pl.when(c + 3 < nchunk)
            def _():
                start(c + 3, vb1, sv1, exc1, se1)
            return 0
        lax.fori_loop(0, nchunk // 2, chunk_pair, 0)
        if nchunk % 2:
            wait(vb0, sv0, exc0, se0)
            compute(nchunk - 1, vb0, exc0)

        plsc.subcore_barrier()
        pltpu.sync_copy(out_sh.at[pl.ds(sid * rows_per_tile, rows_per_tile)],
                        outp_hbm.at[cid, pl.ds(sid * rows_per_tile, rows_per_tile)])

    return pl.kernel(
        body,
        out_type=jax.ShapeDtypeStruct((NC, n_pad, D), F32),
        mesh=_sc_mesh(),
        compiler_params=pltpu.CompilerParams(needs_layout_passes=False),
        scratch_types=[
            pltpu.VMEM((epw,), I32),
            pltpu.VMEM((epw,), I32),
            pltpu.VMEM((b, D), F32),
            pltpu.VMEM((b, D), F32),
            pltpu.VMEM((b,), F32),
            pltpu.VMEM((b,), F32),
            pltpu.SemaphoreType.DMA,
            pltpu.SemaphoreType.DMA,
            pltpu.SemaphoreType.DMA,
            pltpu.SemaphoreType.DMA,
            pltpu.VMEM_SHARED((n_pad, D), F32),
        ],
    )


def kernel(x, edge_index, edge_attr, params):
    n = x.shape[0]
    e = edge_index.shape[1]
    n_pad = ((n + 127) // 128) * 128
    epw = e // NW
    b = 80
    assert epw * NW == e and epw % b == 0

    src = edge_index[0]
    dst = edge_index[1]
    ea = edge_attr[:, 0]

    pass1 = _make_pass1(n, n_pad, e, epw, b)
    pass2 = _make_pass2(n, n_pad, e, epw, b)

    num_layers = len(params['convs'])
    h = x
    for li in range(num_layers):
        p = params['convs'][li]
        c = p['Wq'].shape[1]
        inv_sc = 1.0 / math.sqrt(float(c))
        wcat = jnp.concatenate(
            [p['Wq'] * inv_sc, p['Wk'], p['Wv'], p['Ws']], axis=1)
        bcat = jnp.concatenate(
            [p['bq'] * inv_sc, p['bk'], p['bv'], p['bs']]).reshape(1, 4 * D)
        we = p['We'][0].reshape(1, D)
        be = p['be'].reshape(1, D)
        wb = p['Wbeta'][:, 0]
        g1 = (wb[:D] + wb[2 * D:]).reshape(1, D)
        g2 = (wb[D:2 * D] - wb[2 * D:]).reshape(1, D)

        q, k, v, r = _projections(h, wcat, bcat, 400)
        qwe, qbe2 = _node_scalars(q, k, we, be, n_pad)
        ex, den_p, u_p = pass1(q, k, dst, src, ea, qwe, qbe2)
        dinv, satt, t = _combine(den_p, u_p, n_pad)
        out_p = pass2(v, ex, dst, src)

        if li < num_layers - 1:
            gam = params['norms'][li]['gamma'].reshape(1, D)
            bln = params['norms'][li]['beta'].reshape(1, D)
            h = _gate(out_p, r, dinv, satt, t, we, be, g1, g2, gam, bln, True)
        else:
            h = _gate(out_p, r, dinv, satt, t, we, be, g1, g2, be, be, False)
    return h


# speedup vs baseline: 2.8097x; 1.0040x over previous
import functools
import math

import jax
import jax.numpy as jnp
from jax import lax
from jax.experimental import pallas as pl
from jax.experimental.pallas import tpu as pltpu
from jax.experimental.pallas import tpu_sc as plsc

F32 = jnp.float32
I32 = jnp.int32

NC = 2
NS = 16
NW = NC * NS
LANES = 16

D = 128
SHIFT_MARGIN = 60.0


def _proj_body(h_ref, w_ref, b_ref, q_ref, k_ref, v_ref, r_ref):
    pr = jnp.dot(h_ref[...], w_ref[...], preferred_element_type=F32) + b_ref[...]
    q_ref[...] = pr[:, :D]
    k_ref[...] = pr[:, D:2 * D]
    v_ref[...] = pr[:, 2 * D:3 * D]
    r_ref[...] = pr[:, 3 * D:]


def _projections(h, wcat, bcat, n_blk):
    n = h.shape[0]
    grid = n // n_blk
    out = jax.ShapeDtypeStruct((n, D), F32)
    return pl.pallas_call(
        _proj_body,
        grid=(grid,),
        in_specs=[
            pl.BlockSpec((n_blk, D), lambda i: (i, 0)),
            pl.BlockSpec((D, 4 * D), lambda i: (0, 0)),
            pl.BlockSpec((1, 4 * D), lambda i: (0, 0)),
        ],
        out_specs=[pl.BlockSpec((n_blk, D), lambda i: (i, 0))] * 4,
        out_shape=[out, out, out, out],
    )(h, wcat, bcat)


def _scalars_body(q_ref, k_ref, we_ref, be_ref, qwe_ref, qbe2_ref):
    q = q_ref[...]
    k = k_ref[...]
    we = we_ref[...]
    be = be_ref[...]
    qwe = jnp.sum(q * we, axis=1)
    qbe = jnp.sum(q * be, axis=1)
    qn = jnp.sqrt(jnp.sum(q * q, axis=1))
    kn2max = jnp.max(jnp.sum(k * k, axis=1))
    shift = qn * jnp.sqrt(kn2max) - SHIFT_MARGIN
    pad = jnp.zeros((qwe_ref.shape[0] - q.shape[0],), F32)
    qwe_ref[...] = jnp.concatenate([qwe, pad])
    qbe2_ref[...] = jnp.concatenate([qbe - shift, pad])


def _node_scalars(q, k, we, be, n_pad):
    out = jax.ShapeDtypeStruct((n_pad,), F32)
    return pl.pallas_call(_scalars_body, out_shape=[out, out])(q, k, we, be)


def _combine_body(den_ref, u_ref, dinv_ref, satt_ref, t_ref):
    den = jnp.sum(den_ref[...], axis=0)
    u = jnp.sum(u_ref[...], axis=0)
    pos = den > 0.0
    dinv = jnp.where(pos, 1.0 / jnp.where(pos, den, 1.0), 0.0)
    dinv_ref[...] = dinv
    satt_ref[...] = u * dinv
    t_ref[...] = jnp.where(pos, 1.0, 0.0)


def _combine(den_p, u_p, n_pad):
    out = jax.ShapeDtypeStruct((n_pad,), F32)
    return pl.pallas_call(_combine_body, out_shape=[out, out, out])(den_p, u_p)


def _gate_body(op_ref, r_ref, dinv_ref, satt_ref, t_ref, we_ref, be_ref,
               g1_ref, g2_ref, gam_ref, bln_ref, h_ref, *, do_ln):
    n = r_ref.shape[0]
    o = op_ref[0, :n, :] + op_ref[1, :n, :]
    di = dinv_ref[...][:n]
    sa = satt_ref[...][:n]
    tt = t_ref[...][:n]
    o = di[:, None] * o + sa[:, None] * we_ref[...] + tt[:, None] * be_ref[...]
    r = r_ref[...]
    logit = (jnp.sum(o * g1_ref[...], axis=1, keepdims=True)
             + jnp.sum(r * g2_ref[...], axis=1, keepdims=True))
    beta = jax.nn.sigmoid(logit)
    h = beta * r + (1.0 - beta) * o
    if do_ln:
        mu = jnp.mean(h, axis=1, keepdims=True)
        var = jnp.mean((h - mu) ** 2, axis=1, keepdims=True)
        h = (h - mu) / jnp.sqrt(var + 1e-5) * gam_ref[...] + bln_ref[...]
        h = jnp.maximum(h, 0.0)
    h_ref[...] = h


def _gate(out_p, r, dinv, satt, t, we, be, g1, g2, gam, bln, do_ln):
    n = r.shape[0]
    return pl.pallas_call(
        functools.partial(_gate_body, do_ln=do_ln),
        out_shape=jax.ShapeDtypeStruct((n, D), F32),
    )(out_p, r, dinv, satt, t, we, be, g1, g2, gam, bln)


def _sc_mesh():
    return plsc.VectorSubcoreMesh(core_axis_name="c", subcore_axis_name="s")


def _make_pass1(n, n_pad, e, epw, b):
    nchunk = epw // b
    ngrp = b // LANES

    def body(q_hbm, k_hbm, dst_hbm, src_hbm, ea_hbm, qwe_hbm, qbe2_hbm,
             ex_hbm, den_hbm, u_hbm,
             dst_l, src_l, ea_l, ex_l, qwe_l, qbe2_l, den_l, u_l,
             qb0, kb0, qb1, kb1, sq0, sk0, sq1, sk1):
        wid = lax.axis_index("s") * NC + lax.axis_index("c")
        base = wid * epw
        pltpu.sync_copy(dst_hbm.at[pl.ds(base, epw)], dst_l)
        pltpu.sync_copy(src_hbm.at[pl.ds(base, epw)], src_l)
        pltpu.sync_copy(ea_hbm.at[pl.ds(base, epw)], ea_l)
        pltpu.sync_copy(qwe_hbm, qwe_l)
        pltpu.sync_copy(qbe2_hbm, qbe2_l)

        def zbody(i, _):
            z = jnp.zeros((LANES,), F32)
            den_l[pl.ds(i * LANES, LANES)] = z
            u_l[pl.ds(i * LANES, LANES)] = z
            return 0
        lax.fori_loop(0, n_pad // LANES, zbody, 0, unroll=4)

        def start(c, qb, kb, sq, sk):
            pltpu.make_async_copy(
                q_hbm.at[dst_l.at[pl.ds(c * b, b)]], qb, sq).start()
            pltpu.make_async_copy(
                k_hbm.at[src_l.at[pl.ds(c * b, b)]], kb, sk).start()

        def wait(qb, kb, sq, sk):
            pltpu.make_async_copy(q_hbm.at[pl.ds(0, b)], qb, sq).wait()
            pltpu.make_async_copy(k_hbm.at[pl.ds(0, b)], kb, sk).wait()

        def compute(c, qb, kb):
            for g in range(ngrp):
                goff = c * b + g * LANES
                dstg = dst_l[pl.ds(goff, LANES)]
                eag = ea_l[pl.ds(goff, LANES)]
                qweg = plsc.load_gather(qwe_l, [dstg])
                qbe2g = plsc.load_gather(qbe2_l, [dstg])
                ridx = lax.iota(I32, LANES) + g * LANES

                def jbody(j, acc):
                    a = acc
                    for jj in range(4):
                        jc = jnp.full((LANES,), j * 4 + jj, I32)
                        colq = plsc.load_gather(qb, [ridx, jc])
                        colk = plsc.load_gather(kb, [ridx, jc])
                        a = a + colq * colk
                    return a
                acc = lax.fori_loop(0, D // 4, jbody,
                                    jnp.zeros((LANES,), F32))
                ex = jnp.exp(acc + eag * qweg + qbe2g)
                ex_l[pl.ds(goff, LANES)] = ex
                exu = ex * eag
                plsc.addupdate_scatter(den_l, [dstg], ex)
                plsc.addupdate_scatter(u_l, [dstg], exu)

        start(0, qb0, kb0, sq0, sk0)
        start(1, qb1, kb1, sq1, sk1)

        def chunk_pair(i, _):
            c = i * 2
            wait(qb0, kb0, sq0, sk0)
            compute(c, qb0, kb0)

            @pl.when(c + 2 < nchunk)
            def _():
                start(c + 2, qb0, kb0, sq0, sk0)
            wait(qb1, kb1, sq1, sk1)
            compute(c + 1, qb1, kb1)

            @pl.when(c + 3 < nchunk)
            def _():
                start(c + 3, qb1, kb1, sq1, sk1)
            return 0
        lax.fori_loop(0, nchunk // 2, chunk_pair, 0)
        if nchunk % 2:
            wait(qb0, kb0, sq0, sk0)
            compute(nchunk - 1, qb0, kb0)

        pltpu.sync_copy(ex_l, ex_hbm.at[pl.ds(base, epw)])
        pltpu.sync_copy(den_l, den_hbm.at[wid])
        pltpu.sync_copy(u_l, u_hbm.at[wid])

    return pl.kernel(
        body,
        out_type=(jax.ShapeDtypeStruct((e,), F32),
                  jax.ShapeDtypeStruct((NW, n_pad), F32),
                  jax.ShapeDtypeStruct((NW, n_pad), F32)),
        mesh=_sc_mesh(),
        compiler_params=pltpu.CompilerParams(needs_layout_passes=False),
        scratch_types=[
            pltpu.VMEM((epw,), I32),
            pltpu.VMEM((epw,), I32),
            pltpu.VMEM((epw,), F32),
            pltpu.VMEM((epw,), F32),
            pltpu.VMEM((n_pad,), F32),
            pltpu.VMEM((n_pad,), F32),
            pltpu.VMEM((n_pad,), F32),
            pltpu.VMEM((n_pad,), F32),
            pltpu.VMEM((b, D), F32),
            pltpu.VMEM((b, D), F32),
            pltpu.VMEM((b, D), F32),
            pltpu.VMEM((b, D), F32),
            pltpu.SemaphoreType.DMA,
            pltpu.SemaphoreType.DMA,
            pltpu.SemaphoreType.DMA,
            pltpu.SemaphoreType.DMA,
        ],
    )


def _make_pass2(n, n_pad, e, epw, b):
    nchunk = epw // b
    ngrp = b // LANES
    rows_per_tile = n_pad // NS

    def body(v_hbm, ex_hbm, dst_hbm, src_hbm,
             outp_hbm,
             dst_l, src_l,
             vb0, vb1, exc0, exc1, sv0, sv1, se0, se1, out_sh):
        cid = lax.axis_index("c")
        sid = lax.axis_index("s")
        wid = sid * NC + cid
        base = wid * epw
        pltpu.sync_copy(dst_hbm.at[pl.ds(base, epw)], dst_l)
        pltpu.sync_copy(src_hbm.at[pl.ds(base, epw)], src_l)

        def zb(i, _):
            vb0[i // (D // LANES), pl.ds((i % (D // LANES)) * LANES, LANES)] = (
                jnp.zeros((LANES,), F32))
            return 0
        lax.fori_loop(0, b * D // LANES, zb, 0, unroll=4)
        for j in range(rows_per_tile // b):
            pltpu.sync_copy(vb0, out_sh.at[pl.ds(sid * rows_per_tile + j * b, b)])
        rem = rows_per_tile % b
        if rem:
            pltpu.sync_copy(
                vb0.at[pl.ds(0, rem)],
                out_sh.at[pl.ds(sid * rows_per_tile + (rows_per_tile // b) * b,
                                rem)])
        plsc.subcore_barrier()

        def start(c, vb, sv, exc, se):
            pltpu.make_async_copy(
                v_hbm.at[src_l.at[pl.ds(c * b, b)]], vb, sv).start()
            pltpu.make_async_copy(
                ex_hbm.at[pl.ds(base + c * b, b)], exc, se).start()

        def wait(vb, sv, exc, se):
            pltpu.make_async_copy(v_hbm.at[pl.ds(0, b)], vb, sv).wait()
            pltpu.make_async_copy(ex_hbm.at[pl.ds(0, b)], exc, se).wait()

        def compute(c, vb, exc):
            ws = [exc[pl.ds(g * LANES, LANES)] for g in range(ngrp)]

            def jbody(j, _):
                for g in range(ngrp):
                    ridx = lax.iota(I32, LANES) + g * LANES
                    jc = jnp.full((LANES,), j, I32)
                    col = plsc.load_gather(vb, [ridx, jc])
                    plsc.store_scatter(vb, [ridx, jc], col * ws[g])
                return 0
            lax.fori_loop(0, D, jbody, 0, unroll=2)
            pltpu.sync_copy(vb, out_sh.at[dst_l.at[pl.ds(c * b, b)]], add=True)

        start(0, vb0, sv0, exc0, se0)
        start(1, vb1, sv1, exc1, se1)

        def chunk_pair(i, _):
            c = i * 2
            wait(vb0, sv0, exc0, se0)
            compute(c, vb0, exc0)

            @pl.when(c + 2 < nchunk)
            def _():
                start(c + 2, vb0, sv0, exc0, se0)
            wait(vb1, sv1, exc1, se1)
            compute(c + 1, vb1, exc1)

            @pl.when(c + 3 < nchunk)
            def _():
                start(c + 3, vb1, sv1, exc1, se1)
            return 0
        lax.fori_loop(0, nchunk // 2, chunk_pair, 0)
        if nchunk % 2:
            wait(vb0, sv0, exc0, se0)
            compute(nchunk - 1, vb0, exc0)

        plsc.subcore_barrier()
        pltpu.sync_copy(out_sh.at[pl.ds(sid * rows_per_tile, rows_per_tile)],
                        outp_hbm.at[cid, pl.ds(sid * rows_per_tile, rows_per_tile)])

    return pl.kernel(
        body,
        out_type=jax.ShapeDtypeStruct((NC, n_pad, D), F32),
        mesh=_sc_mesh(),
        compiler_params=pltpu.CompilerParams(needs_layout_passes=False),
        scratch_types=[
            pltpu.VMEM((epw,), I32),
            pltpu.VMEM((epw,), I32),
            pltpu.VMEM((b, D), F32),
            pltpu.VMEM((b, D), F32),
            pltpu.VMEM((b,), F32),
            pltpu.VMEM((b,), F32),
            pltpu.SemaphoreType.DMA,
            pltpu.SemaphoreType.DMA,
            pltpu.SemaphoreType.DMA,
            pltpu.SemaphoreType.DMA,
            pltpu.VMEM_SHARED((n_pad, D), F32),
        ],
    )


def kernel(x, edge_index, edge_attr, params):
    n = x.shape[0]
    e = edge_index.shape[1]
    n_pad = ((n + 127) // 128) * 128
    epw = e // NW
    b = 80
    assert epw * NW == e and epw % b == 0

    src = edge_index[0]
    dst = edge_index[1]
    ea = edge_attr[:, 0]

    pass1 = _make_pass1(n, n_pad, e, epw, b)
    pass2 = _make_pass2(n, n_pad, e, epw, b)

    num_layers = len(params['convs'])
    h = x
    for li in range(num_layers):
        p = params['convs'][li]
        c = p['Wq'].shape[1]
        inv_sc = 1.0 / math.sqrt(float(c))
        wcat = jnp.concatenate(
            [p['Wq'] * inv_sc, p['Wk'], p['Wv'], p['Ws']], axis=1)
        bcat = jnp.concatenate(
            [p['bq'] * inv_sc, p['bk'], p['bv'], p['bs']]).reshape(1, 4 * D)
        we = p['We'][0].reshape(1, D)
        be = p['be'].reshape(1, D)
        wb = p['Wbeta'][:, 0]
        g1 = (wb[:D] + wb[2 * D:]).reshape(1, D)
        g2 = (wb[D:2 * D] - wb[2 * D:]).reshape(1, D)

        q, k, v, r = _projections(h, wcat, bcat, 400)
        qwe, qbe2 = _node_scalars(q, k, we, be, n_pad)
        ex, den_p, u_p = pass1(q, k, dst, src, ea, qwe, qbe2)
        dinv, satt, t = _combine(den_p, u_p, n_pad)
        out_p = pass2(v, ex, dst, src)

        if li < num_layers - 1:
            gam = params['norms'][li]['gamma'].reshape(1, D)
            bln = params['norms'][li]['beta'].reshape(1, D)
            h = _gate(out_p, r, dinv, satt, t, we, be, g1, g2, gam, bln, True)
        else:
            h = _gate(out_p, r, dinv, satt, t, we, be, g1, g2, be, be, False)
    return h
